# Initial kernel scaffold; baseline (speedup 1.0000x reference)
import jax
import jax.numpy as jnp
from jax.experimental import pallas as pl


def kernel(state_input, adj):
    raise NotImplementedError("write your pallas kernel here")


# speedup vs baseline: 4.8820x; 4.8820x over previous
import functools

import jax
import jax.numpy as jnp
from jax import lax
from jax.experimental import pallas as pl
from jax.experimental.pallas import tpu as pltpu
from jax.experimental.pallas import tpu_sc as plsc

NC = 2
NS = 16
B = 128


def _sc_segsum(state, row_r, col_r, zblk):
    n, d = state.shape
    nb = row_r.shape[2]
    rows_per_sub = -(-(n + 1) // (NS * B)) * B
    acc_rows = NS * rows_per_sub
    nz = rows_per_sub // B

    mesh = plsc.VectorSubcoreMesh(core_axis_name="c", subcore_axis_name="s")

    @functools.partial(
        pl.kernel,
        out_type=jax.ShapeDtypeStruct((NC, acc_rows, d), jnp.float32),
        mesh=mesh,
        scratch_types=[
            pltpu.VMEM((nb, B), jnp.int32),
            pltpu.VMEM((nb, B), jnp.int32),
            pltpu.VMEM((B, d), jnp.float32),
            pltpu.VMEM_SHARED((acc_rows, d), jnp.float32),
            pltpu.SemaphoreType.DMA,
        ],
    )
    def k(state_hbm, row_hbm, col_hbm, z_hbm, out_hbm, row_v, col_v, buf,
          acc, sem):
        cid = lax.axis_index("c")
        sid = lax.axis_index("s")
        pltpu.sync_copy(row_hbm.at[cid, sid], row_v)
        pltpu.sync_copy(col_hbm.at[cid, sid], col_v)
        pltpu.sync_copy(z_hbm, buf)
        for t in range(nz):
            pltpu.sync_copy(
                buf, acc.at[pl.ds(sid * rows_per_sub + t * B, B)])
        plsc.subcore_barrier()

        def body(j, carry):
            pltpu.async_copy(state_hbm.at[row_v.at[j]], buf, sem).wait()
            pltpu.sync_copy(buf, acc.at[col_v.at[j]], add=True)
            return carry

        lax.fori_loop(0, nb, body, 0)
        plsc.subcore_barrier()
        for t in range(nz):
            r0 = sid * rows_per_sub + t * B
            pltpu.sync_copy(acc.at[pl.ds(r0, B)], buf)
            pltpu.sync_copy(buf, out_hbm.at[cid, pl.ds(r0, B)])

    return k(state, row_r, col_r, zblk)


def _combine(partials, n):
    d = partials.shape[2]
    rb = 1000

    def body(p_ref, o_ref):
        o_ref[...] = p_ref[0] + p_ref[1]

    return pl.pallas_call(
        body,
        grid=(n // rb,),
        in_specs=[pl.BlockSpec((2, rb, d), lambda i: (0, i, 0))],
        out_specs=pl.BlockSpec((rb, d), lambda i: (i, 0)),
        out_shape=jax.ShapeDtypeStruct((n, d), jnp.float32),
    )(partials)


def kernel(state_input, adj):
    n, d = state_input.shape
    e = adj.shape[1]
    nb = -(-e // (NC * NS * B))
    pad = NC * NS * nb * B - e
    row = adj[0]
    col = adj[1]
    row_p = jnp.concatenate([row, jnp.zeros((pad,), jnp.int32)])
    col_p = jnp.concatenate([col, jnp.full((pad,), n, jnp.int32)])
    row_r = row_p.reshape(NC, NS, nb, B)
    col_r = col_p.reshape(NC, NS, nb, B)
    zblk = jnp.zeros((B, d), jnp.float32)
    partials = _sc_segsum(state_input, row_r, col_r, zblk)
    out = _combine(partials, n)
    zeros = jnp.zeros_like(state_input)
    return (out, state_input, zeros, out, out)


# speedup vs baseline: 6.3114x; 1.2928x over previous
import functools

import jax
import jax.numpy as jnp
from jax import lax
from jax.experimental import pallas as pl
from jax.experimental.pallas import tpu as pltpu
from jax.experimental.pallas import tpu_sc as plsc

NC = 2
NS = 16
B = 112
NBUF = 3


def _sc_segsum(state, idx_r, zblk):
    n, d = state.shape
    nb = idx_r.shape[2]
    rows_per_sub = -(-(n + 1) // (NS * 8)) * 8
    acc_rows = NS * rows_per_sub
    chunks = []
    r = 0
    while r < rows_per_sub:
        c = min(B, rows_per_sub - r)
        chunks.append((r, c))
        r += c

    mesh = plsc.VectorSubcoreMesh(core_axis_name="c", subcore_axis_name="s")

    @functools.partial(
        pl.kernel,
        out_type=jax.ShapeDtypeStruct((NC, acc_rows, d), jnp.float32),
        mesh=mesh,
        scratch_types=(
            [pltpu.VMEM((2, B), jnp.int32)] * NBUF
            + [pltpu.VMEM((B, d), jnp.float32)] * NBUF
            + [pltpu.VMEM_SHARED((acc_rows, d), jnp.float32)]
            + [pltpu.SemaphoreType.DMA] * (2 * NBUF)
        ),
    )
    def k(state_hbm, idx_hbm, z_hbm, out_hbm, *rest):
        ibufs = rest[:NBUF]
        bufs = rest[NBUF:2 * NBUF]
        acc = rest[2 * NBUF]
        isems = rest[2 * NBUF + 1:3 * NBUF + 1]
        gsems = rest[3 * NBUF + 1:]
        cid = lax.axis_index("c")
        sid = lax.axis_index("s")
        pltpu.sync_copy(z_hbm, bufs[0])
        for r0, c in chunks:
            pltpu.sync_copy(bufs[0].at[pl.ds(0, c)],
                            acc.at[pl.ds(sid * rows_per_sub + r0, c)])
        plsc.subcore_barrier()

        for b in range(NBUF):
            pltpu.async_copy(idx_hbm.at[cid, sid, b], ibufs[b], isems[b])
        for b in range(NBUF - 1):
            pltpu.make_async_copy(
                idx_hbm.at[cid, sid, b], ibufs[b], isems[b]).wait()
            pltpu.async_copy(
                state_hbm.at[ibufs[b].at[0]], bufs[b], gsems[b])

        def step(jg, ji, b):
            bp = (b + NBUF - 1) % NBUF
            if jg is not None:
                pltpu.make_async_copy(
                    idx_hbm.at[cid, sid, jg], ibufs[bp], isems[bp]).wait()
                pltpu.async_copy(
                    state_hbm.at[ibufs[bp].at[0]], bufs[bp], gsems[bp])
            pltpu.make_async_copy(
                state_hbm.at[ibufs[b].at[0]], bufs[b], gsems[b]).wait()
            pltpu.sync_copy(bufs[b], acc.at[ibufs[b].at[1]], add=True)
            if ji is not None:
                pltpu.async_copy(idx_hbm.at[cid, sid, ji], ibufs[b], isems[b])

        def body(i, carry):
            for b in range(NBUF):
                j = i * NBUF + b
                step(j + NBUF - 1, j + NBUF, b)
            return carry

        lax.fori_loop(0, nb // NBUF - 1, body, 0)
        for b in range(NBUF):
            step(nb - 1 if b == 0 else None, None, b)
        plsc.subcore_barrier()
        for t, (r0, c) in enumerate(chunks):
            b = t % NBUF
            r = sid * rows_per_sub + r0
            pltpu.sync_copy(acc.at[pl.ds(r, c)], bufs[b].at[pl.ds(0, c)])
            pltpu.sync_copy(bufs[b].at[pl.ds(0, c)],
                            out_hbm.at[cid, pl.ds(r, c)])

    return k(state, idx_r, zblk)


def _combine(partials, n):
    d = partials.shape[2]
    rb = 1000

    def body(p_ref, o_ref):
        o_ref[...] = p_ref[0] + p_ref[1]

    return pl.pallas_call(
        body,
        grid=(n // rb,),
        in_specs=[pl.BlockSpec((2, rb, d), lambda i: (0, i, 0))],
        out_specs=pl.BlockSpec((rb, d), lambda i: (i, 0)),
        out_shape=jax.ShapeDtypeStruct((n, d), jnp.float32),
    )(partials)


def kernel(state_input, adj):
    n, d = state_input.shape
    e = adj.shape[1]
    nb = -(-e // (NC * NS * B))
    nb = -(-nb // NBUF) * NBUF
    pad = NC * NS * nb * B - e
    row_p = jnp.concatenate([adj[0], jnp.zeros((pad,), jnp.int32)])
    col_p = jnp.concatenate([adj[1], jnp.full((pad,), n, jnp.int32)])
    idx_r = jnp.stack(
        [row_p.reshape(NC, NS, nb, B), col_p.reshape(NC, NS, nb, B)], axis=3)
    zblk = jnp.zeros((B, d), jnp.float32)
    partials = _sc_segsum(state_input, idx_r, zblk)
    out = _combine(partials, n)
    zeros = jnp.zeros_like(state_input)
    return (out, state_input, zeros, out, out)
